# baseline (speedup 1.0000x reference)
import functools

import jax
import jax.numpy as jnp
from jax import lax
from jax.experimental import pallas as pl
from jax.experimental.pallas import tpu as pltpu
from jax.experimental.pallas import tpu_sc as plsc


TB = 512
ST = 32
KC = 256


def _tc1_block(z_ref, win_ref, bin_ref, ct_ref, cnorm_ref,
               ze_ref, idx_ref, *, n_codes):
    nchunk = n_codes // KC

    z_e = lax.dot_general(z_ref[...], win_ref[...],
                          (((1,), (1,)), ((), ())),
                          preferred_element_type=jnp.float32)
    z_e = z_e + bin_ref[...]
    ze_ref[...] = z_e

    lane_iota = lax.broadcasted_iota(jnp.int32, (ST, KC), 1)

    for t in range(TB // ST):
        zet = z_e[t * ST:(t + 1) * ST, :]
        znorm = jnp.sum(zet * zet, axis=1, keepdims=True)
        zem2 = zet * (-2.0)

        run_min = None
        run_chunk = None
        for j in range(nchunk):
            ct_c = ct_ref[:, j * KC:(j + 1) * KC]
            cn_c = cnorm_ref[:, j * KC:(j + 1) * KC]
            s = lax.dot_general(zem2, ct_c,
                                (((1,), (0,)), ((), ())),
                                preferred_element_type=jnp.float32)
            d = (znorm + cn_c) + s
            if j == 0:
                run_min = d
                run_chunk = jnp.zeros((ST, KC), jnp.int32)
            else:
                upd = d < run_min
                run_min = jnp.where(upd, d, run_min)
                run_chunk = jnp.where(upd, jnp.int32(j), run_chunk)

        run_idx = run_chunk * KC + lane_iota
        gmin = jnp.min(run_min, axis=1, keepdims=True)
        idx = jnp.min(jnp.where(run_min == gmin, run_idx, n_codes),
                      axis=1, keepdims=True)
        idx_ref[t * ST:(t + 1) * ST, :] = idx


def _tc2_block(zq_ref, ze_ref, wout_ref, bout_ref,
               out_ref, loss_ref, *, n_dim):
    i = pl.program_id(0)
    z_q = zq_ref[:, :n_dim]

    out_ref[...] = lax.dot_general(z_q, wout_ref[...],
                                   (((1,), (1,)), ((), ())),
                                   preferred_element_type=jnp.float32
                                   ) + bout_ref[...]

    diff = ze_ref[...] - z_q
    part = jnp.sum(diff * diff).reshape(1, 1)

    @pl.when(i == 0)
    def _():
        loss_ref[...] = jnp.zeros_like(loss_ref)

    loss_ref[...] += part


def _sc_gather(table, idx):
    V, D = table.shape
    B = idx.shape[0]
    info = plsc.get_sparse_core_info()
    nw = info.num_cores * info.num_subcores
    per_w = B // nw
    CB = 128
    mesh = plsc.VectorSubcoreMesh(core_axis_name="c", subcore_axis_name="s")

    @functools.partial(
        pl.kernel, mesh=mesh,
        out_type=jax.ShapeDtypeStruct((B, D), jnp.float32),
        compiler_params=pltpu.CompilerParams(use_tc_tiling_on_sc=False),
        scratch_types=[
            pltpu.VMEM((CB,), jnp.int32),
            pltpu.VMEM((CB, D), jnp.float32),
            pltpu.SemaphoreType.DMA,
        ],
    )
    def k(table_hbm, idx_hbm, out_hbm, idx_v, rows_v, sem):
        wid = lax.axis_index("s") * info.num_cores + lax.axis_index("c")
        base = wid * per_w
        for c in range(per_w // CB):
            off = base + c * CB
            pltpu.sync_copy(idx_hbm.at[pl.ds(off, CB)], idx_v)
            pltpu.async_copy(table_hbm.at[idx_v], rows_v, sem).wait()
            pltpu.sync_copy(rows_v, out_hbm.at[pl.ds(off, CB)])

    return k(table, idx)


def kernel(z, W_in, b_in, W_out, b_out, codebook):
    B, N, D = z.shape
    K, C = codebook.shape
    T = B * N
    nblk = T // TB

    z_flat = z.reshape(T, D)
    ct = codebook.T
    cnorm = jnp.sum(codebook ** 2, axis=-1)[None, :]

    z_e, idx = pl.pallas_call(
        functools.partial(_tc1_block, n_codes=K),
        grid=(nblk,),
        in_specs=[
            pl.BlockSpec((TB, D), lambda i: (i, 0)),
            pl.BlockSpec((C, D), lambda i: (0, 0)),
            pl.BlockSpec((1, C), lambda i: (0, 0)),
            pl.BlockSpec((C, K), lambda i: (0, 0)),
            pl.BlockSpec((1, K), lambda i: (0, 0)),
        ],
        out_specs=[
            pl.BlockSpec((TB, C), lambda i: (i, 0)),
            pl.BlockSpec((TB, 1), lambda i: (i, 0)),
        ],
        out_shape=[
            jax.ShapeDtypeStruct((T, C), jnp.float32),
            jax.ShapeDtypeStruct((T, 1), jnp.int32),
        ],
    )(z_flat, W_in, b_in.reshape(1, C), ct, cnorm)

    cb_pad = jnp.pad(codebook, ((0, 0), (0, 8)))
    z_q16 = _sc_gather(cb_pad, idx.reshape(T))

    zq_out, loss_sum = pl.pallas_call(
        functools.partial(_tc2_block, n_dim=C),
        grid=(nblk,),
        in_specs=[
            pl.BlockSpec((TB, 16), lambda i: (i, 0)),
            pl.BlockSpec((TB, C), lambda i: (i, 0)),
            pl.BlockSpec((D, C), lambda i: (0, 0)),
            pl.BlockSpec((1, D), lambda i: (0, 0)),
        ],
        out_specs=[
            pl.BlockSpec((TB, D), lambda i: (i, 0)),
            pl.BlockSpec((1, 1), lambda i: (0, 0)),
        ],
        out_shape=[
            jax.ShapeDtypeStruct((T, D), jnp.float32),
            jax.ShapeDtypeStruct((1, 1), jnp.float32),
        ],
    )(z_q16, z_e, W_out, b_out.reshape(1, D))

    z_q_out = zq_out.reshape(B, N, D)
    indices = idx.reshape(B, N)
    loss = loss_sum[0, 0] / (T * C)
    return (z_q_out, indices, loss, loss)


# speedup vs baseline: 1.0150x; 1.0150x over previous
import functools

import jax
import jax.numpy as jnp
from jax import lax
from jax.experimental import pallas as pl
from jax.experimental.pallas import tpu as pltpu
from jax.experimental.pallas import tpu_sc as plsc


TB = 512
ST = 32
KC = 128


def _tc1_block(z_ref, win_ref, bin_ref, ct_ref, cnorm_ref,
               ze_ref, idx_ref, *, n_codes):
    nchunk = n_codes // KC

    z_e = lax.dot_general(z_ref[...], win_ref[...],
                          (((1,), (1,)), ((), ())),
                          preferred_element_type=jnp.float32)
    z_e = z_e + bin_ref[...]
    ze_ref[...] = z_e

    lane_iota = lax.broadcasted_iota(jnp.int32, (ST, KC), 1)

    for t in range(TB // ST):
        zet = z_e[t * ST:(t + 1) * ST, :]
        znorm = jnp.sum(zet * zet, axis=1, keepdims=True)
        zem2 = zet * (-2.0)

        s_full = lax.dot_general(zem2, ct_ref[...],
                                 (((1,), (0,)), ((), ())),
                                 preferred_element_type=jnp.float32)

        run_min = None
        run_chunk = None
        for j in range(nchunk):
            cn_c = cnorm_ref[:, j * KC:(j + 1) * KC]
            s = s_full[:, j * KC:(j + 1) * KC]
            d = (znorm + cn_c) + s
            if j == 0:
                run_min = d
                run_chunk = jnp.zeros((ST, KC), jnp.int32)
            else:
                upd = d < run_min
                run_min = jnp.where(upd, d, run_min)
                run_chunk = jnp.where(upd, jnp.int32(j), run_chunk)

        run_idx = run_chunk * KC + lane_iota
        gmin = jnp.min(run_min, axis=1, keepdims=True)
        idx = jnp.min(jnp.where(run_min == gmin, run_idx, n_codes),
                      axis=1, keepdims=True)
        idx_ref[t * ST:(t + 1) * ST, :] = idx


def _tc2_block(zq_ref, ze_ref, wout_ref, bout_ref,
               out_ref, loss_ref, *, n_dim):
    i = pl.program_id(0)
    z_q = zq_ref[:, :n_dim]

    out_ref[...] = lax.dot_general(z_q, wout_ref[...],
                                   (((1,), (1,)), ((), ())),
                                   preferred_element_type=jnp.float32
                                   ) + bout_ref[...]

    diff = ze_ref[...] - z_q
    part = jnp.sum(diff * diff).reshape(1, 1)

    @pl.when(i == 0)
    def _():
        loss_ref[...] = jnp.zeros_like(loss_ref)

    loss_ref[...] += part


def _sc_gather(table, idx):
    V, D = table.shape
    B = idx.shape[0]
    info = plsc.get_sparse_core_info()
    nw = info.num_cores * info.num_subcores
    per_w = B // nw
    CB = 128
    mesh = plsc.VectorSubcoreMesh(core_axis_name="c", subcore_axis_name="s")

    @functools.partial(
        pl.kernel, mesh=mesh,
        out_type=jax.ShapeDtypeStruct((B, D), jnp.float32),
        compiler_params=pltpu.CompilerParams(use_tc_tiling_on_sc=False),
        scratch_types=[
            pltpu.VMEM((CB,), jnp.int32),
            pltpu.VMEM((CB, D), jnp.float32),
            pltpu.SemaphoreType.DMA,
        ],
    )
    def k(table_hbm, idx_hbm, out_hbm, idx_v, rows_v, sem):
        wid = lax.axis_index("s") * info.num_cores + lax.axis_index("c")
        base = wid * per_w
        for c in range(per_w // CB):
            off = base + c * CB
            pltpu.sync_copy(idx_hbm.at[pl.ds(off, CB)], idx_v)
            pltpu.async_copy(table_hbm.at[idx_v], rows_v, sem).wait()
            pltpu.sync_copy(rows_v, out_hbm.at[pl.ds(off, CB)])

    return k(table, idx)


def kernel(z, W_in, b_in, W_out, b_out, codebook):
    B, N, D = z.shape
    K, C = codebook.shape
    T = B * N
    nblk = T // TB

    z_flat = z.reshape(T, D)
    ct = codebook.T
    cnorm = jnp.sum(codebook ** 2, axis=-1)[None, :]

    z_e, idx = pl.pallas_call(
        functools.partial(_tc1_block, n_codes=K),
        grid=(nblk,),
        in_specs=[
            pl.BlockSpec((TB, D), lambda i: (i, 0)),
            pl.BlockSpec((C, D), lambda i: (0, 0)),
            pl.BlockSpec((1, C), lambda i: (0, 0)),
            pl.BlockSpec((C, K), lambda i: (0, 0)),
            pl.BlockSpec((1, K), lambda i: (0, 0)),
        ],
        out_specs=[
            pl.BlockSpec((TB, C), lambda i: (i, 0)),
            pl.BlockSpec((TB, 1), lambda i: (i, 0)),
        ],
        out_shape=[
            jax.ShapeDtypeStruct((T, C), jnp.float32),
            jax.ShapeDtypeStruct((T, 1), jnp.int32),
        ],
    )(z_flat, W_in, b_in.reshape(1, C), ct, cnorm)

    cb_pad = jnp.pad(codebook, ((0, 0), (0, 8)))
    z_q16 = _sc_gather(cb_pad, idx.reshape(T))

    zq_out, loss_sum = pl.pallas_call(
        functools.partial(_tc2_block, n_dim=C),
        grid=(nblk,),
        in_specs=[
            pl.BlockSpec((TB, 16), lambda i: (i, 0)),
            pl.BlockSpec((TB, C), lambda i: (i, 0)),
            pl.BlockSpec((D, C), lambda i: (0, 0)),
            pl.BlockSpec((1, D), lambda i: (0, 0)),
        ],
        out_specs=[
            pl.BlockSpec((TB, D), lambda i: (i, 0)),
            pl.BlockSpec((1, 1), lambda i: (0, 0)),
        ],
        out_shape=[
            jax.ShapeDtypeStruct((T, D), jnp.float32),
            jax.ShapeDtypeStruct((1, 1), jnp.float32),
        ],
    )(z_q16, z_e, W_out, b_out.reshape(1, D))

    z_q_out = zq_out.reshape(B, N, D)
    indices = idx.reshape(B, N)
    loss = loss_sum[0, 0] / (T * C)
    return (z_q_out, indices, loss, loss)


# speedup vs baseline: 1.0755x; 1.0596x over previous
import functools

import jax
import jax.numpy as jnp
from jax import lax
from jax.experimental import pallas as pl
from jax.experimental.pallas import tpu as pltpu
from jax.experimental.pallas import tpu_sc as plsc


TB = 1024
ST = 32
KC = 128


def _tc1_block(z_ref, win_ref, bin_ref, ct_ref, cnorm_ref,
               ze_ref, idx_ref, *, n_codes):
    nchunk = n_codes // KC

    z_e = lax.dot_general(z_ref[...], win_ref[...],
                          (((1,), (1,)), ((), ())),
                          preferred_element_type=jnp.float32)
    z_e = z_e + bin_ref[...]
    ze_ref[...] = z_e

    lane_iota = lax.broadcasted_iota(jnp.int32, (ST, KC), 1)

    for t in range(TB // ST):
        zet = z_e[t * ST:(t + 1) * ST, :]
        znorm = jnp.sum(zet * zet, axis=1, keepdims=True)
        zem2 = zet * (-2.0)

        s_full = lax.dot_general(zem2, ct_ref[...],
                                 (((1,), (0,)), ((), ())),
                                 preferred_element_type=jnp.float32)

        run_min = None
        run_chunk = None
        for j in range(nchunk):
            cn_c = cnorm_ref[:, j * KC:(j + 1) * KC]
            s = s_full[:, j * KC:(j + 1) * KC]
            d = (znorm + cn_c) + s
            if j == 0:
                run_min = d
                run_chunk = jnp.zeros((ST, KC), jnp.int32)
            else:
                upd = d < run_min
                run_min = jnp.where(upd, d, run_min)
                run_chunk = jnp.where(upd, jnp.int32(j), run_chunk)

        run_idx = run_chunk * KC + lane_iota
        gmin = jnp.min(run_min, axis=1, keepdims=True)
        idx = jnp.min(jnp.where(run_min == gmin, run_idx, n_codes),
                      axis=1, keepdims=True)
        idx_ref[t * ST:(t + 1) * ST, :] = idx


def _tc2_block(zq_ref, ze_ref, wout_ref, bout_ref,
               out_ref, loss_ref, *, n_dim):
    i = pl.program_id(0)
    z_q = zq_ref[:, :n_dim]

    out_ref[...] = lax.dot_general(z_q, wout_ref[...],
                                   (((1,), (1,)), ((), ())),
                                   preferred_element_type=jnp.float32
                                   ) + bout_ref[...]

    diff = ze_ref[...] - z_q
    part = jnp.sum(diff * diff).reshape(1, 1)

    @pl.when(i == 0)
    def _():
        loss_ref[...] = jnp.zeros_like(loss_ref)

    loss_ref[...] += part


def _sc_gather(table, idx):
    V, D = table.shape
    B = idx.shape[0]
    info = plsc.get_sparse_core_info()
    nw = info.num_cores * info.num_subcores
    per_w = B // nw
    CB = 128
    mesh = plsc.VectorSubcoreMesh(core_axis_name="c", subcore_axis_name="s")

    @functools.partial(
        pl.kernel, mesh=mesh,
        out_type=jax.ShapeDtypeStruct((B, D), jnp.float32),
        compiler_params=pltpu.CompilerParams(use_tc_tiling_on_sc=False),
        scratch_types=[
            pltpu.VMEM((CB,), jnp.int32),
            pltpu.VMEM((CB, D), jnp.float32),
            pltpu.SemaphoreType.DMA,
        ],
    )
    def k(table_hbm, idx_hbm, out_hbm, idx_v, rows_v, sem):
        wid = lax.axis_index("s") * info.num_cores + lax.axis_index("c")
        base = wid * per_w
        for c in range(per_w // CB):
            off = base + c * CB
            pltpu.sync_copy(idx_hbm.at[pl.ds(off, CB)], idx_v)
            pltpu.async_copy(table_hbm.at[idx_v], rows_v, sem).wait()
            pltpu.sync_copy(rows_v, out_hbm.at[pl.ds(off, CB)])

    return k(table, idx)


def kernel(z, W_in, b_in, W_out, b_out, codebook):
    B, N, D = z.shape
    K, C = codebook.shape
    T = B * N
    nblk = T // TB

    z_flat = z.reshape(T, D)
    ct = codebook.T
    cnorm = jnp.sum(codebook ** 2, axis=-1)[None, :]

    z_e, idx = pl.pallas_call(
        functools.partial(_tc1_block, n_codes=K),
        grid=(nblk,),
        in_specs=[
            pl.BlockSpec((TB, D), lambda i: (i, 0)),
            pl.BlockSpec((C, D), lambda i: (0, 0)),
            pl.BlockSpec((1, C), lambda i: (0, 0)),
            pl.BlockSpec((C, K), lambda i: (0, 0)),
            pl.BlockSpec((1, K), lambda i: (0, 0)),
        ],
        out_specs=[
            pl.BlockSpec((TB, C), lambda i: (i, 0)),
            pl.BlockSpec((TB, 1), lambda i: (i, 0)),
        ],
        out_shape=[
            jax.ShapeDtypeStruct((T, C), jnp.float32),
            jax.ShapeDtypeStruct((T, 1), jnp.int32),
        ],
    )(z_flat, W_in, b_in.reshape(1, C), ct, cnorm)

    cb_pad = jnp.pad(codebook, ((0, 0), (0, 8)))
    z_q16 = _sc_gather(cb_pad, idx.reshape(T))

    zq_out, loss_sum = pl.pallas_call(
        functools.partial(_tc2_block, n_dim=C),
        grid=(nblk,),
        in_specs=[
            pl.BlockSpec((TB, 16), lambda i: (i, 0)),
            pl.BlockSpec((TB, C), lambda i: (i, 0)),
            pl.BlockSpec((D, C), lambda i: (0, 0)),
            pl.BlockSpec((1, D), lambda i: (0, 0)),
        ],
        out_specs=[
            pl.BlockSpec((TB, D), lambda i: (i, 0)),
            pl.BlockSpec((1, 1), lambda i: (0, 0)),
        ],
        out_shape=[
            jax.ShapeDtypeStruct((T, D), jnp.float32),
            jax.ShapeDtypeStruct((1, 1), jnp.float32),
        ],
    )(z_q16, z_e, W_out, b_out.reshape(1, D))

    z_q_out = zq_out.reshape(B, N, D)
    indices = idx.reshape(B, N)
    loss = loss_sum[0, 0] / (T * C)
    return (z_q_out, indices, loss, loss)


# speedup vs baseline: 1.0995x; 1.0223x over previous
import functools

import jax
import jax.numpy as jnp
from jax import lax
from jax.experimental import pallas as pl
from jax.experimental.pallas import tpu as pltpu
from jax.experimental.pallas import tpu_sc as plsc


TB = 2048
ST = 32
KC = 128


def _tc1_block(z_ref, win_ref, bin_ref, ct_ref, cnorm_ref,
               ze_ref, idx_ref, *, n_codes):
    nchunk = n_codes // KC

    z_e = lax.dot_general(z_ref[...], win_ref[...],
                          (((1,), (1,)), ((), ())),
                          preferred_element_type=jnp.float32)
    z_e = z_e + bin_ref[...]
    ze_ref[...] = z_e

    lane_iota = lax.broadcasted_iota(jnp.int32, (ST, KC), 1)

    for t in range(TB // ST):
        zet = z_e[t * ST:(t + 1) * ST, :]
        znorm = jnp.sum(zet * zet, axis=1, keepdims=True)
        zem2 = zet * (-2.0)

        s_full = lax.dot_general(zem2, ct_ref[...],
                                 (((1,), (0,)), ((), ())),
                                 preferred_element_type=jnp.float32)

        run_min = None
        run_chunk = None
        for j in range(nchunk):
            cn_c = cnorm_ref[:, j * KC:(j + 1) * KC]
            s = s_full[:, j * KC:(j + 1) * KC]
            d = (znorm + cn_c) + s
            if j == 0:
                run_min = d
                run_chunk = jnp.zeros((ST, KC), jnp.int32)
            else:
                upd = d < run_min
                run_min = jnp.where(upd, d, run_min)
                run_chunk = jnp.where(upd, jnp.int32(j), run_chunk)

        run_idx = run_chunk * KC + lane_iota
        gmin = jnp.min(run_min, axis=1, keepdims=True)
        idx = jnp.min(jnp.where(run_min == gmin, run_idx, n_codes),
                      axis=1, keepdims=True)
        idx_ref[t * ST:(t + 1) * ST, :] = idx


def _tc2_block(zq_ref, ze_ref, wout_ref, bout_ref,
               out_ref, loss_ref, *, n_dim):
    i = pl.program_id(0)
    z_q = zq_ref[:, :n_dim]

    out_ref[...] = lax.dot_general(z_q, wout_ref[...],
                                   (((1,), (1,)), ((), ())),
                                   preferred_element_type=jnp.float32
                                   ) + bout_ref[...]

    diff = ze_ref[...] - z_q
    part = jnp.sum(diff * diff).reshape(1, 1)

    @pl.when(i == 0)
    def _():
        loss_ref[...] = jnp.zeros_like(loss_ref)

    loss_ref[...] += part


def _sc_gather(table, idx):
    V, D = table.shape
    B = idx.shape[0]
    info = plsc.get_sparse_core_info()
    nw = info.num_cores * info.num_subcores
    per_w = B // nw
    CB = 128
    mesh = plsc.VectorSubcoreMesh(core_axis_name="c", subcore_axis_name="s")

    @functools.partial(
        pl.kernel, mesh=mesh,
        out_type=jax.ShapeDtypeStruct((B, D), jnp.float32),
        compiler_params=pltpu.CompilerParams(use_tc_tiling_on_sc=False),
        scratch_types=[
            pltpu.VMEM((CB,), jnp.int32),
            pltpu.VMEM((CB, D), jnp.float32),
            pltpu.SemaphoreType.DMA,
        ],
    )
    def k(table_hbm, idx_hbm, out_hbm, idx_v, rows_v, sem):
        wid = lax.axis_index("s") * info.num_cores + lax.axis_index("c")
        base = wid * per_w
        for c in range(per_w // CB):
            off = base + c * CB
            pltpu.sync_copy(idx_hbm.at[pl.ds(off, CB)], idx_v)
            pltpu.async_copy(table_hbm.at[idx_v], rows_v, sem).wait()
            pltpu.sync_copy(rows_v, out_hbm.at[pl.ds(off, CB)])

    return k(table, idx)


def kernel(z, W_in, b_in, W_out, b_out, codebook):
    B, N, D = z.shape
    K, C = codebook.shape
    T = B * N
    nblk = T // TB

    z_flat = z.reshape(T, D)
    ct = codebook.T
    cnorm = jnp.sum(codebook ** 2, axis=-1)[None, :]

    z_e, idx = pl.pallas_call(
        functools.partial(_tc1_block, n_codes=K),
        grid=(nblk,),
        in_specs=[
            pl.BlockSpec((TB, D), lambda i: (i, 0)),
            pl.BlockSpec((C, D), lambda i: (0, 0)),
            pl.BlockSpec((1, C), lambda i: (0, 0)),
            pl.BlockSpec((C, K), lambda i: (0, 0)),
            pl.BlockSpec((1, K), lambda i: (0, 0)),
        ],
        out_specs=[
            pl.BlockSpec((TB, C), lambda i: (i, 0)),
            pl.BlockSpec((TB, 1), lambda i: (i, 0)),
        ],
        out_shape=[
            jax.ShapeDtypeStruct((T, C), jnp.float32),
            jax.ShapeDtypeStruct((T, 1), jnp.int32),
        ],
    )(z_flat, W_in, b_in.reshape(1, C), ct, cnorm)

    cb_pad = jnp.pad(codebook, ((0, 0), (0, 8)))
    z_q16 = _sc_gather(cb_pad, idx.reshape(T))

    zq_out, loss_sum = pl.pallas_call(
        functools.partial(_tc2_block, n_dim=C),
        grid=(nblk,),
        in_specs=[
            pl.BlockSpec((TB, 16), lambda i: (i, 0)),
            pl.BlockSpec((TB, C), lambda i: (i, 0)),
            pl.BlockSpec((D, C), lambda i: (0, 0)),
            pl.BlockSpec((1, D), lambda i: (0, 0)),
        ],
        out_specs=[
            pl.BlockSpec((TB, D), lambda i: (i, 0)),
            pl.BlockSpec((1, 1), lambda i: (0, 0)),
        ],
        out_shape=[
            jax.ShapeDtypeStruct((T, D), jnp.float32),
            jax.ShapeDtypeStruct((1, 1), jnp.float32),
        ],
    )(z_q16, z_e, W_out, b_out.reshape(1, D))

    z_q_out = zq_out.reshape(B, N, D)
    indices = idx.reshape(B, N)
    loss = loss_sum[0, 0] / (T * C)
    return (z_q_out, indices, loss, loss)


# speedup vs baseline: 1.1041x; 1.0042x over previous
import functools

import jax
import jax.numpy as jnp
from jax import lax
from jax.experimental import pallas as pl
from jax.experimental.pallas import tpu as pltpu
from jax.experimental.pallas import tpu_sc as plsc


TB = 4096
ST = 32
KC = 128


def _tc1_block(z_ref, win_ref, bin_ref, ct_ref, cnorm_ref,
               ze_ref, idx_ref, *, n_codes):
    nchunk = n_codes // KC

    z_e = lax.dot_general(z_ref[...], win_ref[...],
                          (((1,), (1,)), ((), ())),
                          preferred_element_type=jnp.float32)
    z_e = z_e + bin_ref[...]
    ze_ref[...] = z_e

    lane_iota = lax.broadcasted_iota(jnp.int32, (ST, KC), 1)

    for t in range(TB // ST):
        zet = z_e[t * ST:(t + 1) * ST, :]
        znorm = jnp.sum(zet * zet, axis=1, keepdims=True)
        zem2 = zet * (-2.0)

        s_full = lax.dot_general(zem2, ct_ref[...],
                                 (((1,), (0,)), ((), ())),
                                 preferred_element_type=jnp.float32)

        run_min = None
        run_chunk = None
        for j in range(nchunk):
            cn_c = cnorm_ref[:, j * KC:(j + 1) * KC]
            s = s_full[:, j * KC:(j + 1) * KC]
            d = (znorm + cn_c) + s
            if j == 0:
                run_min = d
                run_chunk = jnp.zeros((ST, KC), jnp.int32)
            else:
                upd = d < run_min
                run_min = jnp.where(upd, d, run_min)
                run_chunk = jnp.where(upd, jnp.int32(j), run_chunk)

        run_idx = run_chunk * KC + lane_iota
        gmin = jnp.min(run_min, axis=1, keepdims=True)
        idx = jnp.min(jnp.where(run_min == gmin, run_idx, n_codes),
                      axis=1, keepdims=True)
        idx_ref[t * ST:(t + 1) * ST, :] = idx


def _tc2_block(zq_ref, ze_ref, wout_ref, bout_ref,
               out_ref, loss_ref, *, n_dim):
    i = pl.program_id(0)
    z_q = zq_ref[:, :n_dim]

    out_ref[...] = lax.dot_general(z_q, wout_ref[...],
                                   (((1,), (1,)), ((), ())),
                                   preferred_element_type=jnp.float32
                                   ) + bout_ref[...]

    diff = ze_ref[...] - z_q
    part = jnp.sum(diff * diff).reshape(1, 1)

    @pl.when(i == 0)
    def _():
        loss_ref[...] = jnp.zeros_like(loss_ref)

    loss_ref[...] += part


def _sc_gather(table, idx):
    V, D = table.shape
    B = idx.shape[0]
    info = plsc.get_sparse_core_info()
    nw = info.num_cores * info.num_subcores
    per_w = B // nw
    CB = 128
    mesh = plsc.VectorSubcoreMesh(core_axis_name="c", subcore_axis_name="s")

    @functools.partial(
        pl.kernel, mesh=mesh,
        out_type=jax.ShapeDtypeStruct((B, D), jnp.float32),
        compiler_params=pltpu.CompilerParams(use_tc_tiling_on_sc=False),
        scratch_types=[
            pltpu.VMEM((CB,), jnp.int32),
            pltpu.VMEM((CB, D), jnp.float32),
            pltpu.SemaphoreType.DMA,
        ],
    )
    def k(table_hbm, idx_hbm, out_hbm, idx_v, rows_v, sem):
        wid = lax.axis_index("s") * info.num_cores + lax.axis_index("c")
        base = wid * per_w
        for c in range(per_w // CB):
            off = base + c * CB
            pltpu.sync_copy(idx_hbm.at[pl.ds(off, CB)], idx_v)
            pltpu.async_copy(table_hbm.at[idx_v], rows_v, sem).wait()
            pltpu.sync_copy(rows_v, out_hbm.at[pl.ds(off, CB)])

    return k(table, idx)


def kernel(z, W_in, b_in, W_out, b_out, codebook):
    B, N, D = z.shape
    K, C = codebook.shape
    T = B * N
    nblk = T // TB

    z_flat = z.reshape(T, D)
    ct = codebook.T
    cnorm = jnp.sum(codebook ** 2, axis=-1)[None, :]

    z_e, idx = pl.pallas_call(
        functools.partial(_tc1_block, n_codes=K),
        grid=(nblk,),
        in_specs=[
            pl.BlockSpec((TB, D), lambda i: (i, 0)),
            pl.BlockSpec((C, D), lambda i: (0, 0)),
            pl.BlockSpec((1, C), lambda i: (0, 0)),
            pl.BlockSpec((C, K), lambda i: (0, 0)),
            pl.BlockSpec((1, K), lambda i: (0, 0)),
        ],
        out_specs=[
            pl.BlockSpec((TB, C), lambda i: (i, 0)),
            pl.BlockSpec((TB, 1), lambda i: (i, 0)),
        ],
        out_shape=[
            jax.ShapeDtypeStruct((T, C), jnp.float32),
            jax.ShapeDtypeStruct((T, 1), jnp.int32),
        ],
    )(z_flat, W_in, b_in.reshape(1, C), ct, cnorm)

    cb_pad = jnp.pad(codebook, ((0, 0), (0, 8)))
    z_q16 = _sc_gather(cb_pad, idx.reshape(T))

    zq_out, loss_sum = pl.pallas_call(
        functools.partial(_tc2_block, n_dim=C),
        grid=(nblk,),
        in_specs=[
            pl.BlockSpec((TB, 16), lambda i: (i, 0)),
            pl.BlockSpec((TB, C), lambda i: (i, 0)),
            pl.BlockSpec((D, C), lambda i: (0, 0)),
            pl.BlockSpec((1, D), lambda i: (0, 0)),
        ],
        out_specs=[
            pl.BlockSpec((TB, D), lambda i: (i, 0)),
            pl.BlockSpec((1, 1), lambda i: (0, 0)),
        ],
        out_shape=[
            jax.ShapeDtypeStruct((T, D), jnp.float32),
            jax.ShapeDtypeStruct((1, 1), jnp.float32),
        ],
    )(z_q16, z_e, W_out, b_out.reshape(1, D))

    z_q_out = zq_out.reshape(B, N, D)
    indices = idx.reshape(B, N)
    loss = loss_sum[0, 0] / (T * C)
    return (z_q_out, indices, loss, loss)
